# baseline (speedup 1.0000x reference)
import functools
import math

import jax
import jax.numpy as jnp
from jax import lax
from jax.experimental import pallas as pl
from jax.experimental.pallas import tpu as pltpu
from jax.experimental.pallas import tpu_sc as plsc

D = 64
SCALE = math.sqrt(D)

_info = plsc.get_sparse_core_info()
NC, NS, L = _info.num_cores, _info.num_subcores, _info.num_lanes
NW = NC * NS

CH = 128


@functools.lru_cache(maxsize=None)
def _make(B):
    b_per_w = B // NW
    n_chunks = b_per_w // CH
    mesh = plsc.VectorSubcoreMesh(core_axis_name="c", subcore_axis_name="s")

    @functools.partial(
        pl.kernel,
        mesh=mesh,
        out_type=jax.ShapeDtypeStruct((B, D), jnp.float32),
        scratch_types=[
            pltpu.VMEM((CH,), jnp.int32),
            pltpu.VMEM((CH, D), jnp.float32),
            pltpu.SemaphoreType.DMA,
        ],
        compiler_params=pltpu.CompilerParams(use_tc_tiling_on_sc=False),
    )
    def k(x_hbm, table_hbm, out_hbm, idx_v, rows_v, sem):
        wid = lax.axis_index("s") * NC + lax.axis_index("c")
        base = wid * b_per_w

        def chunk_body(g, carry):
            off = pl.multiple_of(base + g * CH, CH)
            pltpu.sync_copy(x_hbm.at[pl.ds(off, CH)], idx_v)
            pltpu.async_copy(table_hbm.at[idx_v], rows_v, sem).wait()

            def row_body(r, c):
                for j in range(D // L):
                    v = rows_v[r, pl.ds(j * L, L)]
                    rows_v[r, pl.ds(j * L, L)] = v * SCALE
                return c

            lax.fori_loop(0, CH, row_body, 0)
            pltpu.sync_copy(rows_v, out_hbm.at[pl.ds(off, CH)])
            return carry

        lax.fori_loop(0, n_chunks, chunk_body, 0)

    return k


@jax.jit
def kernel(x, embedding):
    s0, s1 = x.shape
    B = s0 * s1
    idx = x.reshape(-1).astype(jnp.int32)
    out = _make(B)(idx, embedding)
    return out.reshape(s0, s1, D)


# speedup vs baseline: 1.2471x; 1.2471x over previous
import functools
import math

import jax
import jax.numpy as jnp
from jax import lax
from jax.experimental import pallas as pl
from jax.experimental.pallas import tpu as pltpu
from jax.experimental.pallas import tpu_sc as plsc

D = 64
SCALE = math.sqrt(D)

_info = plsc.get_sparse_core_info()
NC, NS, L = _info.num_cores, _info.num_subcores, _info.num_lanes
NW = NC * NS

CH = 128
NBUF = 4
UR = 8


@functools.lru_cache(maxsize=None)
def _make(B):
    b_per_w = B // NW
    G = b_per_w // CH
    assert G % NBUF == 0
    mesh = plsc.VectorSubcoreMesh(core_axis_name="c", subcore_axis_name="s")

    @functools.partial(
        pl.kernel,
        mesh=mesh,
        out_type=jax.ShapeDtypeStruct((B, D), jnp.float32),
        scratch_types=[
            pltpu.VMEM((G, CH), jnp.int32),
            [pltpu.VMEM((CH, D), jnp.float32) for _ in range(NBUF)],
            [pltpu.VMEM((CH, D), jnp.float32) for _ in range(NBUF)],
            [pltpu.SemaphoreType.DMA for _ in range(NBUF)],
            [pltpu.SemaphoreType.DMA for _ in range(NBUF)],
        ],
        compiler_params=pltpu.CompilerParams(use_tc_tiling_on_sc=False),
    )
    def k(x_hbm, table_hbm, out_hbm, idx_v, gbufs, wbufs, gsems, wsems):
        wid = lax.axis_index("s") * NC + lax.axis_index("c")
        base = wid * b_per_w

        pltpu.sync_copy(x_hbm.at[wid], idx_v)

        def g_desc(b, g):
            return pltpu.make_async_copy(
                table_hbm.at[idx_v.at[g]], gbufs[b], gsems[b])

        def w_desc(b, g):
            return pltpu.make_async_copy(
                wbufs[b], out_hbm.at[pl.ds(base + g * CH, CH)], wsems[b])

        def scale(gb, wb):
            def row_body(r, c):
                for rr in range(UR):
                    row = r * UR + rr
                    for j in range(D // L):
                        v = gb[row, pl.ds(j * L, L)]
                        wb[row, pl.ds(j * L, L)] = v * SCALE
                return c

            lax.fori_loop(0, CH // UR, row_body, 0, unroll=True)

        for b in range(NBUF):
            g_desc(b, b).start()

        def rnd(i, c):
            s = i * NBUF
            for b in range(NBUF):
                g = s + b
                g_desc(b, g).wait()

                @pl.when(s > 0)
                def _():
                    w_desc(b, g - NBUF).wait()

                scale(gbufs[b], wbufs[b])

                @pl.when(g + NBUF < G)
                def _():
                    g_desc(b, g + NBUF).start()

                w_desc(b, g).start()
            return c

        lax.fori_loop(0, G // NBUF, rnd, 0)

        for b in range(NBUF):
            w_desc(b, G - NBUF + b).wait()

    return k


@jax.jit
def kernel(x, embedding):
    s0, s1 = x.shape
    B = s0 * s1
    idx = x.reshape(NW, (B // NW) // CH, CH).astype(jnp.int32)
    out = _make(B)(idx, embedding)
    return out.reshape(s0, s1, D)


# speedup vs baseline: 1.5498x; 1.2427x over previous
import functools
import math

import jax
import jax.numpy as jnp
from jax import lax
from jax.experimental import pallas as pl
from jax.experimental.pallas import tpu as pltpu
from jax.experimental.pallas import tpu_sc as plsc

D = 64
DP = 128
SCALE = math.sqrt(D)

_info = plsc.get_sparse_core_info()
NC, NS, L = _info.num_cores, _info.num_subcores, _info.num_lanes
NW = NC * NS

CH = 128
NBUF = 4
NWB = 2
UR = 8


@functools.lru_cache(maxsize=None)
def _make(B):
    b_per_w = B // NW
    G = b_per_w // CH
    assert G % NBUF == 0
    mesh = plsc.VectorSubcoreMesh(core_axis_name="c", subcore_axis_name="s")

    @functools.partial(
        pl.kernel,
        mesh=mesh,
        out_type=jax.ShapeDtypeStruct((B, D), jnp.float32),
        scratch_types=[
            pltpu.VMEM((G, CH), jnp.int32),
            [pltpu.VMEM((CH, DP), jnp.float32) for _ in range(NBUF)],
            [pltpu.VMEM((CH, D), jnp.float32) for _ in range(NWB)],
            [pltpu.SemaphoreType.DMA for _ in range(NBUF)],
            [pltpu.SemaphoreType.DMA for _ in range(NWB)],
        ],
    )
    def k(x_hbm, table_hbm, out_hbm, idx_v, gbufs, wbufs, gsems, wsems):
        wid = lax.axis_index("s") * NC + lax.axis_index("c")
        base = wid * b_per_w

        pltpu.sync_copy(x_hbm.at[wid], idx_v)

        def g_desc(b, g):
            return pltpu.make_async_copy(
                table_hbm.at[idx_v.at[g]], gbufs[b], gsems[b])

        def w_desc(b, g):
            return pltpu.make_async_copy(
                wbufs[b % NWB],
                out_hbm.at[pl.ds(base + g * CH, CH)],
                wsems[b % NWB],
            )

        def scale(gb, wb):
            def row_body(r, c):
                for rr in range(UR):
                    row = r * UR + rr
                    for j in range(D // L):
                        v = gb[row, pl.ds(j * L, L)]
                        wb[row, pl.ds(j * L, L)] = v * SCALE
                return c

            lax.fori_loop(0, CH // UR, row_body, 0, unroll=True)

        for b in range(NBUF):
            g_desc(b, b).start()

        def rnd(i, c):
            s = i * NBUF
            for b in range(NBUF):
                g = s + b
                g_desc(b, g).wait()

                @pl.when(g >= NWB)
                def _():
                    w_desc(b, g - NWB).wait()

                scale(gbufs[b], wbufs[b % NWB])

                @pl.when(g + NBUF < G)
                def _():
                    g_desc(b, g + NBUF).start()

                w_desc(b, g).start()
            return c

        lax.fori_loop(0, G // NBUF, rnd, 0)

        for b in range(NWB):
            w_desc(b, G - NWB + b).wait()

    return k


@jax.jit
def kernel(x, embedding):
    s0, s1 = x.shape
    B = s0 * s1
    idx = x.reshape(NW, (B // NW) // CH, CH).astype(jnp.int32)
    table2 = jnp.pad(embedding, ((0, 0), (0, DP - D)))
    out = _make(B)(idx, table2)
    return out.reshape(s0, s1, D)
